# Initial kernel scaffold; baseline (speedup 1.0000x reference)
import jax
import jax.numpy as jnp
from jax.experimental import pallas as pl


def kernel(frame_features, boundaries, ph_labels):
    raise NotImplementedError("write your pallas kernel here")


# speedup vs baseline: 3.4607x; 3.4607x over previous
import jax
import jax.numpy as jnp
from jax import lax
from jax.experimental import pallas as pl
from jax.experimental.pallas import tpu as pltpu
from jax.experimental.pallas import tpu_sc as plsc

_B, _T, _D = 16, 2048, 512
_TOTAL = _B * _T
_NT = 16
_C = _TOTAL // _NT
_MARGIN = 0.3
_EPS = 1e-6
_MASK_CHUNKS = _C // 16
_BITS_PAD = 2080


def _sqrt16(x):
    i = plsc.bitcast(x, jnp.int32)
    y = plsc.bitcast(jnp.int32(0x5F3759DF) - (i >> 1), jnp.float32)
    for _ in range(3):
        y = y * (1.5 - 0.5 * x * y * y)
    return x * y


def _body(ff, bd, ph, hb, lb, out, negl,
          labels_v, bound_v, pos_v, neg_v, hb_v, lb_v,
          negoff_v, cnt_v, splat_v, nidx_v,
          ridx_v, aidx_v, pidx_v, gidx_v,
          abuf, pbuf, nbuf, outf_v, out_v, pf_v,
          counts_s, part_s,
          sem0, sem1, sem2, sem3):
    c_ax = lax.axis_index("c")

    @pl.when(c_ax == 0)
    def _main():
        w = lax.axis_index("s")
        lane = lax.iota(jnp.int32, 16)
        zero16 = jnp.zeros((16,), jnp.int32)

        pltpu.sync_copy(ph.at[w], labels_v.at[pl.ds(0, _C)])
        pltpu.sync_copy(bd.at[w], bound_v)

        base = w * _C

        def p1(c, carry):
            offp, offn = carry
            t0 = c * 16
            labv = labels_v[pl.ds(t0, 16)]
            nxtv = plsc.load_gather(labels_v, [t0 + 1 + lane])
            bndv = bound_v[pl.ds(t0, 16)]
            tvec = t0 + lane
            isv = jnp.logical_and(lax.rem(labv, 3) == 2, labv < 30)
            isn = jnp.logical_and(lax.rem(nxtv, 3) == 2, nxtv < 30)
            m = (bndv > 0.1) & isv & isn & (tvec < _T - 1)
            flat = base + tvec
            mi = m.astype(jnp.int32)
            slot_p = offp + plsc.cumsum(mi) - 1
            slot_n = offn + plsc.cumsum(1 - mi) - 1
            plsc.store_scatter(pos_v, [slot_p], flat, mask=m)
            plsc.store_scatter(neg_v, [slot_n >> 7, slot_n & 127], flat,
                               mask=jnp.logical_not(m))
            cnt = jnp.sum(mi)
            return offp + cnt, offn + (16 - cnt)

        npos_w, _ = lax.fori_loop(0, _MASK_CHUNKS, p1,
                                  (jnp.int32(0), jnp.int32(0)))

        splat_v[...] = jnp.full((16,), npos_w, jnp.int32)
        pltpu.sync_copy(splat_v, counts_s.at[pl.ds(w * 16, 16)])
        pltpu.sync_copy(neg_v.at[pl.ds(0, _C // 128)],
                        negl.at[pl.ds(w * (_C // 128), _C // 128)])
        plsc.subcore_barrier()

        pltpu.sync_copy(counts_s, cnt_v)
        cnts = plsc.load_gather(cnt_v, [lane * 17])
        negc = _C - cnts
        negoff = plsc.cumsum(negc) - negc
        posoff = plsc.cumsum(cnts) - cnts
        n_pos = jnp.sum(cnts)
        n_neg = _TOTAL - n_pos
        posoff_w = jnp.sum(jnp.where(lane == w, posoff, 0))
        negoff_v[...] = negoff

        span = jnp.maximum(n_neg, 1).astype(jnp.uint32)
        mult = lax.rem(jnp.uint32(65536), span)
        mult = lax.rem(mult * mult, span)

        base8 = (posoff_w // 8) * 8
        rem8 = posoff_w - base8
        pltpu.sync_copy(hb.at[pl.ds(base8, 2064)], hb_v)
        pltpu.sync_copy(lb.at[pl.ds(base8, 2064)], lb_v)

        nchunks = (npos_w + 15) // 16

        def p3(c, acc):
            lbase = c * 16
            valid = lane < (npos_w - lbase)
            a = pos_v[pl.ds(lbase, 16)]
            a = jnp.where(valid, a, 0)
            t = a & (_T - 1)
            pvec = jnp.where(t > 0, a - 1, a)
            hbv = plsc.bitcast(
                plsc.load_gather(hb_v, [rem8 + lbase + lane]), jnp.uint32)
            lbv = plsc.bitcast(
                plsc.load_gather(lb_v, [rem8 + lbase + lane]), jnp.uint32)
            r = lax.rem(lax.rem(hbv, span) * mult + lax.rem(lbv, span), span)
            r = r.astype(jnp.int32)
            u = jnp.zeros((16,), jnp.int32)
            for uu in range(_NT):
                u = u + jnp.where(r >= negoff[uu], 1, 0)
            u = u - 1
            local = r - plsc.load_gather(negoff_v, [u])
            addr = u * _C + local
            ridx_v[...] = addr >> 7
            pltpu.async_copy(negl.at[ridx_v], nidx_v, sem0).wait()
            n_idx = plsc.load_gather(nidx_v, [lane, addr & 127])
            n_idx = jnp.where(valid, n_idx, 0)
            aidx_v[...] = a
            pidx_v[...] = pvec
            gidx_v[...] = n_idx
            ca = pltpu.async_copy(ff.at[aidx_v], abuf, sem1)
            cb = pltpu.async_copy(ff.at[pidx_v], pbuf, sem2)
            cc = pltpu.async_copy(ff.at[gidx_v], nbuf, sem3)
            ca.wait()
            cb.wait()
            cc.wait()
            d2a = jnp.zeros((16,), jnp.float32)
            d2b = jnp.zeros((16,), jnp.float32)
            for j in range(16):
                sa = jnp.zeros((16,), jnp.float32)
                sb = jnp.zeros((16,), jnp.float32)
                for k in range(_D // 16):
                    av = abuf[j, pl.ds(k * 16, 16)]
                    pv = pbuf[j, pl.ds(k * 16, 16)]
                    nv = nbuf[j, pl.ds(k * 16, 16)]
                    da = av - pv + _EPS
                    db = av - nv + _EPS
                    sa = sa + da * da
                    sb = sb + db * db
                d2a = jnp.where(lane == j, jnp.sum(sa), d2a)
                d2b = jnp.where(lane == j, jnp.sum(sb), d2b)
            dap = _sqrt16(d2a)
            dan = _sqrt16(d2b)
            per = jnp.maximum(dap - dan + _MARGIN, 0.0)
            return acc + jnp.where(valid, per, 0.0)

        acc = lax.fori_loop(0, nchunks, p3, jnp.zeros((16,), jnp.float32))

        outf_v[...] = jnp.full((16,), jnp.sum(acc), jnp.float32)
        pltpu.sync_copy(outf_v, part_s.at[pl.ds(w * 16, 16)])
        plsc.subcore_barrier()

        @pl.when(w == 0)
        def _():
            pltpu.sync_copy(part_s, pf_v)
            tot = jnp.sum(plsc.load_gather(pf_v, [lane * 17]))
            totv = jnp.full((16,), tot, jnp.float32)
            denv = jnp.full((16,), jnp.maximum(n_pos, 1).astype(jnp.float32),
                            jnp.float32)
            lossv = totv / denv
            fallback = (n_pos == 0) | (n_neg == 0)
            out_v[...] = jnp.where(fallback, jnp.float32(1e-6), lossv)
            pltpu.sync_copy(out_v, out)


def _make_call():
    mesh = plsc.VectorSubcoreMesh(core_axis_name="c", subcore_axis_name="s",
                                  num_cores=2, num_subcores=16)
    return pl.kernel(
        _body,
        out_type=(jax.ShapeDtypeStruct((16,), jnp.float32),
                  jax.ShapeDtypeStruct((_TOTAL // 128, 128), jnp.int32)),
        mesh=mesh,
        compiler_params=pltpu.CompilerParams(needs_layout_passes=False),
        scratch_types=[
            pltpu.VMEM((2064,), jnp.int32),
            pltpu.VMEM((2048,), jnp.float32),
            pltpu.VMEM((2064,), jnp.int32),
            pltpu.VMEM((17, 128), jnp.int32),
            pltpu.VMEM((2064,), jnp.int32),
            pltpu.VMEM((2064,), jnp.int32),
            pltpu.VMEM((16,), jnp.int32),
            pltpu.VMEM((256,), jnp.int32),
            pltpu.VMEM((16,), jnp.int32),
            pltpu.VMEM((16, 128), jnp.int32),
            pltpu.VMEM((16,), jnp.int32),
            pltpu.VMEM((16,), jnp.int32),
            pltpu.VMEM((16,), jnp.int32),
            pltpu.VMEM((16,), jnp.int32),
            pltpu.VMEM((16, _D), jnp.float32),
            pltpu.VMEM((16, _D), jnp.float32),
            pltpu.VMEM((16, _D), jnp.float32),
            pltpu.VMEM((16,), jnp.float32),
            pltpu.VMEM((16,), jnp.float32),
            pltpu.VMEM((256,), jnp.float32),
            pltpu.VMEM_SHARED((256,), jnp.int32),
            pltpu.VMEM_SHARED((256,), jnp.float32),
            pltpu.SemaphoreType.DMA,
            pltpu.SemaphoreType.DMA,
            pltpu.SemaphoreType.DMA,
            pltpu.SemaphoreType.DMA,
        ],
    )


_sc_call = _make_call()


def kernel(frame_features, boundaries, ph_labels):
    ff = frame_features.reshape(_TOTAL, _D)
    k1, k2 = jax.random.split(jax.random.key(123))
    zpad = jnp.zeros((_BITS_PAD,), jnp.uint32)
    hb = lax.bitcast_convert_type(
        jnp.concatenate([jax.random.bits(k1, (_TOTAL,), jnp.uint32), zpad]),
        jnp.int32)
    lb = lax.bitcast_convert_type(
        jnp.concatenate([jax.random.bits(k2, (_TOTAL,), jnp.uint32), zpad]),
        jnp.int32)
    out, _ = _sc_call(ff, boundaries, ph_labels.astype(jnp.int32), hb, lb)
    return out[0]


# speedup vs baseline: 3.8772x; 1.1203x over previous
import jax
import jax.numpy as jnp
from jax import lax
from jax.experimental import pallas as pl
from jax.experimental.pallas import tpu as pltpu
from jax.experimental.pallas import tpu_sc as plsc

_B, _T, _D = 16, 2048, 512
_TOTAL = _B * _T
_NT = 16
_C = _TOTAL // _NT
_MARGIN = 0.3
_EPS = 1e-6
_MASK_CHUNKS = _C // 16
_BITS_PAD = 2080


def _sqrt16(x):
    i = plsc.bitcast(x, jnp.int32)
    y = plsc.bitcast(jnp.int32(0x5F3759DF) - (i >> 1), jnp.float32)
    for _ in range(3):
        y = y * (1.5 - 0.5 * x * y * y)
    return x * y


def _body(ff, bd, ph, hb, lb, out, negl,
          labels_v, bound_v, pos_v, neg_v, hb_v, lb_v,
          negoff_v, cnt_v, splat_v, nidx_v,
          ridx_v, aidx_va, pidx_va, gidx_va, aidx_vb, pidx_vb, gidx_vb,
          abuf_a, pbuf_a, nbuf_a, abuf_b, pbuf_b, nbuf_b,
          d2sa, d2sb, outf_v, out_v, pf_v,
          counts_s, part_s,
          sem0, sem1a, sem2a, sem3a, sem1b, sem2b, sem3b):
    c_ax = lax.axis_index("c")

    @pl.when(c_ax == 0)
    def _main():
        w = lax.axis_index("s")
        lane = lax.iota(jnp.int32, 16)

        pltpu.sync_copy(ph.at[w], labels_v.at[pl.ds(0, _C)])
        pltpu.sync_copy(bd.at[w], bound_v)

        base = w * _C

        def p1(c, carry):
            offp, offn = carry
            t0 = c * 16
            labv = labels_v[pl.ds(t0, 16)]
            nxtv = plsc.load_gather(labels_v, [t0 + 1 + lane])
            bndv = bound_v[pl.ds(t0, 16)]
            tvec = t0 + lane
            isv = jnp.logical_and(lax.rem(labv, 3) == 2, labv < 30)
            isn = jnp.logical_and(lax.rem(nxtv, 3) == 2, nxtv < 30)
            m = (bndv > 0.1) & isv & isn & (tvec < _T - 1)
            flat = base + tvec
            mi = m.astype(jnp.int32)
            cs = plsc.cumsum(mi)
            slot_p = offp + cs - 1
            slot_n = offn + lane - cs
            plsc.store_scatter(pos_v, [slot_p], flat, mask=m)
            plsc.store_scatter(neg_v, [slot_n >> 7, slot_n & 127], flat,
                               mask=jnp.logical_not(m))
            cnt = cs[15]
            return offp + cnt, offn + (16 - cnt)

        npos_w, _ = lax.fori_loop(0, _MASK_CHUNKS, p1,
                                  (jnp.int32(0), jnp.int32(0)))

        splat_v[...] = jnp.full((16,), npos_w, jnp.int32)
        pltpu.sync_copy(splat_v, counts_s.at[pl.ds(w * 16, 16)])
        pltpu.sync_copy(neg_v.at[pl.ds(0, _C // 128)],
                        negl.at[pl.ds(w * (_C // 128), _C // 128)])
        plsc.subcore_barrier()

        pltpu.sync_copy(counts_s, cnt_v)
        cnts = plsc.load_gather(cnt_v, [lane * 17])
        negc = _C - cnts
        negoff = plsc.cumsum(negc) - negc
        posoff = plsc.cumsum(cnts) - cnts
        n_pos = jnp.sum(cnts)
        n_neg = _TOTAL - n_pos
        posoff_w = jnp.sum(jnp.where(lane == w, posoff, 0))
        negoff_v[...] = negoff

        span = jnp.maximum(n_neg, 1).astype(jnp.uint32)
        mult = lax.rem(jnp.uint32(65536), span)
        mult = lax.rem(mult * mult, span)

        base8 = (posoff_w // 8) * 8
        rem8 = posoff_w - base8
        pltpu.sync_copy(hb.at[pl.ds(base8, 2064)], hb_v)
        pltpu.sync_copy(lb.at[pl.ds(base8, 2064)], lb_v)

        nchunks = (npos_w + 15) // 16

        def start_chunk(c, aidx_v, pidx_v, gidx_v, ab, pb, nb, s1, s2, s3):
            @pl.when(c < nchunks)
            def _():
                lbase = c * 16
                valid = lane < (npos_w - lbase)
                a = pos_v[pl.ds(lbase, 16)]
                a = jnp.where(valid, a, 0)
                t = a & (_T - 1)
                pvec = jnp.where(t > 0, a - 1, a)
                hbv = plsc.bitcast(
                    plsc.load_gather(hb_v, [rem8 + lbase + lane]), jnp.uint32)
                lbv = plsc.bitcast(
                    plsc.load_gather(lb_v, [rem8 + lbase + lane]), jnp.uint32)
                r = lax.rem(lax.rem(hbv, span) * mult + lax.rem(lbv, span),
                            span)
                r = r.astype(jnp.int32)
                u = jnp.zeros((16,), jnp.int32)
                for uu in range(_NT):
                    u = u + jnp.where(r >= negoff[uu], 1, 0)
                u = u - 1
                local = r - plsc.load_gather(negoff_v, [u])
                addr = u * _C + local
                ridx_v[...] = addr >> 7
                pltpu.async_copy(negl.at[ridx_v], nidx_v, sem0).wait()
                n_idx = plsc.load_gather(nidx_v, [lane, addr & 127])
                n_idx = jnp.where(valid, n_idx, 0)
                aidx_v[...] = a
                pidx_v[...] = pvec
                gidx_v[...] = n_idx
                pltpu.make_async_copy(ff.at[aidx_v], ab, s1).start()
                pltpu.make_async_copy(ff.at[pidx_v], pb, s2).start()
                pltpu.make_async_copy(ff.at[gidx_v], nb, s3).start()

        def consume_chunk(c, ab, pb, nb, s1, s2, s3, acc):
            @pl.when(c < nchunks)
            def _():
                pltpu.make_async_copy(ff.at[lane], ab, s1).wait()
                pltpu.make_async_copy(ff.at[lane], pb, s2).wait()
                pltpu.make_async_copy(ff.at[lane], nb, s3).wait()
            lbase = c * 16
            valid = lane < (npos_w - lbase)

            def drow(j, _):
                sa = jnp.zeros((16,), jnp.float32)
                sb = jnp.zeros((16,), jnp.float32)
                for k in range(_D // 16):
                    av = ab[j, pl.ds(k * 16, 16)]
                    pv = pb[j, pl.ds(k * 16, 16)]
                    nv = nb[j, pl.ds(k * 16, 16)]
                    da = av - pv + _EPS
                    db = av - nv + _EPS
                    sa = sa + da * da
                    sb = sb + db * db
                d2sa[pl.ds(j * 16, 16)] = sa
                d2sb[pl.ds(j * 16, 16)] = sb
                return 0

            lax.fori_loop(0, 16, drow, 0)
            d2a = jnp.zeros((16,), jnp.float32)
            d2b = jnp.zeros((16,), jnp.float32)
            for col in range(16):
                d2a = d2a + plsc.load_gather(d2sa, [lane * 16 + col])
                d2b = d2b + plsc.load_gather(d2sb, [lane * 16 + col])
            dap = _sqrt16(d2a)
            dan = _sqrt16(d2b)
            per = jnp.maximum(dap - dan + _MARGIN, 0.0)
            return acc + jnp.where(valid, per, 0.0)

        start_chunk(0, aidx_va, pidx_va, gidx_va,
                    abuf_a, pbuf_a, nbuf_a, sem1a, sem2a, sem3a)

        def gbody(g, acc):
            c0 = 2 * g
            start_chunk(c0 + 1, aidx_vb, pidx_vb, gidx_vb,
                        abuf_b, pbuf_b, nbuf_b, sem1b, sem2b, sem3b)
            acc = consume_chunk(c0, abuf_a, pbuf_a, nbuf_a,
                                sem1a, sem2a, sem3a, acc)
            start_chunk(c0 + 2, aidx_va, pidx_va, gidx_va,
                        abuf_a, pbuf_a, nbuf_a, sem1a, sem2a, sem3a)
            acc = consume_chunk(c0 + 1, abuf_b, pbuf_b, nbuf_b,
                                sem1b, sem2b, sem3b, acc)
            return acc

        npairs = (nchunks + 1) // 2
        acc = lax.fori_loop(0, npairs, gbody, jnp.zeros((16,), jnp.float32))

        outf_v[...] = jnp.full((16,), jnp.sum(acc), jnp.float32)
        pltpu.sync_copy(outf_v, part_s.at[pl.ds(w * 16, 16)])
        plsc.subcore_barrier()

        @pl.when(w == 0)
        def _():
            pltpu.sync_copy(part_s, pf_v)
            tot = jnp.sum(plsc.load_gather(pf_v, [lane * 17]))
            totv = jnp.full((16,), tot, jnp.float32)
            denv = jnp.full((16,), jnp.maximum(n_pos, 1).astype(jnp.float32),
                            jnp.float32)
            lossv = totv / denv
            fallback = (n_pos == 0) | (n_neg == 0)
            out_v[...] = jnp.where(fallback, jnp.float32(1e-6), lossv)
            pltpu.sync_copy(out_v, out)


def _make_call():
    mesh = plsc.VectorSubcoreMesh(core_axis_name="c", subcore_axis_name="s",
                                  num_cores=2, num_subcores=16)
    return pl.kernel(
        _body,
        out_type=(jax.ShapeDtypeStruct((16,), jnp.float32),
                  jax.ShapeDtypeStruct((_TOTAL // 128, 128), jnp.int32)),
        mesh=mesh,
        compiler_params=pltpu.CompilerParams(needs_layout_passes=False),
        scratch_types=[
            pltpu.VMEM((2064,), jnp.int32),
            pltpu.VMEM((2048,), jnp.float32),
            pltpu.VMEM((2064,), jnp.int32),
            pltpu.VMEM((17, 128), jnp.int32),
            pltpu.VMEM((2064,), jnp.int32),
            pltpu.VMEM((2064,), jnp.int32),
            pltpu.VMEM((16,), jnp.int32),
            pltpu.VMEM((256,), jnp.int32),
            pltpu.VMEM((16,), jnp.int32),
            pltpu.VMEM((16, 128), jnp.int32),
            pltpu.VMEM((16,), jnp.int32),
            pltpu.VMEM((16,), jnp.int32),
            pltpu.VMEM((16,), jnp.int32),
            pltpu.VMEM((16,), jnp.int32),
            pltpu.VMEM((16,), jnp.int32),
            pltpu.VMEM((16,), jnp.int32),
            pltpu.VMEM((16,), jnp.int32),
            pltpu.VMEM((16, _D), jnp.float32),
            pltpu.VMEM((16, _D), jnp.float32),
            pltpu.VMEM((16, _D), jnp.float32),
            pltpu.VMEM((16, _D), jnp.float32),
            pltpu.VMEM((16, _D), jnp.float32),
            pltpu.VMEM((16, _D), jnp.float32),
            pltpu.VMEM((256,), jnp.float32),
            pltpu.VMEM((256,), jnp.float32),
            pltpu.VMEM((16,), jnp.float32),
            pltpu.VMEM((16,), jnp.float32),
            pltpu.VMEM((256,), jnp.float32),
            pltpu.VMEM_SHARED((256,), jnp.int32),
            pltpu.VMEM_SHARED((256,), jnp.float32),
            pltpu.SemaphoreType.DMA,
            pltpu.SemaphoreType.DMA,
            pltpu.SemaphoreType.DMA,
            pltpu.SemaphoreType.DMA,
            pltpu.SemaphoreType.DMA,
            pltpu.SemaphoreType.DMA,
            pltpu.SemaphoreType.DMA,
        ],
    )


_sc_call = _make_call()


def kernel(frame_features, boundaries, ph_labels):
    ff = frame_features.reshape(_TOTAL, _D)
    k1, k2 = jax.random.split(jax.random.key(123))
    zpad = jnp.zeros((_BITS_PAD,), jnp.uint32)
    hb = lax.bitcast_convert_type(
        jnp.concatenate([jax.random.bits(k1, (_TOTAL,), jnp.uint32), zpad]),
        jnp.int32)
    lb = lax.bitcast_convert_type(
        jnp.concatenate([jax.random.bits(k2, (_TOTAL,), jnp.uint32), zpad]),
        jnp.int32)
    out, _ = _sc_call(ff, boundaries, ph_labels.astype(jnp.int32), hb, lb)
    return out[0]


# speedup vs baseline: 3.9802x; 1.0266x over previous
import jax
import jax.numpy as jnp
from jax import lax
from jax.experimental import pallas as pl
from jax.experimental.pallas import tpu as pltpu
from jax.experimental.pallas import tpu_sc as plsc

_B, _T, _D = 16, 2048, 512
_TOTAL = _B * _T
_NT = 16
_C = _TOTAL // _NT
_MARGIN = 0.3
_EPS = 1e-6
_MASK_CHUNKS = _C // 16
_BITS_PAD = 2080


def _sqrt16(x):
    i = plsc.bitcast(x, jnp.int32)
    y = plsc.bitcast(jnp.int32(0x5F3759DF) - (i >> 1), jnp.float32)
    for _ in range(3):
        y = y * (1.5 - 0.5 * x * y * y)
    return x * y


def _body(ff, bd, ph, hb, lb, out, negl,
          labels_v, bound_v, pos_v, neg_v, hb_v, lb_v,
          negoff_v, cnt_v, splat_v, nidx_v,
          ridx_v, aidx_va, pidx_va, gidx_va, aidx_vb, pidx_vb, gidx_vb,
          abuf_a, pbuf_a, nbuf_a, abuf_b, pbuf_b, nbuf_b,
          d2sa, d2sb, outf_v, out_v, pf_v,
          counts_s, part_s,
          sem0, sem1a, sem2a, sem3a, sem1b, sem2b, sem3b):
    c_ax = lax.axis_index("c")

    @pl.when(c_ax == 0)
    def _main():
        w = lax.axis_index("s")
        lane = lax.iota(jnp.int32, 16)

        pltpu.sync_copy(ph.at[w], labels_v.at[pl.ds(0, _C)])
        pltpu.sync_copy(bd.at[w], bound_v)

        base = w * _C

        def p1(c, carry):
            offp, offn = carry
            for h in range(4):
                t0 = c * 64 + h * 16
                labv = labels_v[pl.ds(t0, 16)]
                nxtv = plsc.load_gather(labels_v, [t0 + 1 + lane])
                bndv = bound_v[pl.ds(t0, 16)]
                tvec = t0 + lane
                isv = jnp.logical_and(lax.rem(labv, 3) == 2, labv < 30)
                isn = jnp.logical_and(lax.rem(nxtv, 3) == 2, nxtv < 30)
                m = (bndv > 0.1) & isv & isn & (tvec < _T - 1)
                flat = base + tvec
                mi = m.astype(jnp.int32)
                cs = plsc.cumsum(mi)
                slot_p = offp + cs - 1
                slot_n = offn + lane - cs
                plsc.store_scatter(pos_v, [slot_p], flat, mask=m)
                plsc.store_scatter(neg_v, [slot_n >> 7, slot_n & 127], flat,
                                   mask=jnp.logical_not(m))
                cnt = cs[15]
                offp = offp + cnt
                offn = offn + (16 - cnt)
            return offp, offn

        npos_w, _ = lax.fori_loop(0, _MASK_CHUNKS // 4, p1,
                                  (jnp.int32(0), jnp.int32(0)))

        splat_v[...] = jnp.full((16,), npos_w, jnp.int32)
        pltpu.sync_copy(splat_v, counts_s.at[pl.ds(w * 16, 16)])
        pltpu.sync_copy(neg_v.at[pl.ds(0, _C // 128)],
                        negl.at[pl.ds(w * (_C // 128), _C // 128)])
        plsc.subcore_barrier()

        pltpu.sync_copy(counts_s, cnt_v)
        cnts = plsc.load_gather(cnt_v, [lane * 17])
        negc = _C - cnts
        negoff = plsc.cumsum(negc) - negc
        posoff = plsc.cumsum(cnts) - cnts
        n_pos = jnp.sum(cnts)
        n_neg = _TOTAL - n_pos
        posoff_w = jnp.sum(jnp.where(lane == w, posoff, 0))
        negoff_v[...] = negoff

        span = jnp.maximum(n_neg, 1).astype(jnp.uint32)
        mult = lax.rem(jnp.uint32(65536), span)
        mult = lax.rem(mult * mult, span)

        base8 = (posoff_w // 8) * 8
        rem8 = posoff_w - base8
        pltpu.sync_copy(hb.at[pl.ds(base8, 2064)], hb_v)
        pltpu.sync_copy(lb.at[pl.ds(base8, 2064)], lb_v)

        nchunks = (npos_w + 15) // 16

        def start_chunk(c, aidx_v, pidx_v, gidx_v, ab, pb, nb, s1, s2, s3):
            @pl.when(c < nchunks)
            def _():
                lbase = c * 16
                valid = lane < (npos_w - lbase)
                a = pos_v[pl.ds(lbase, 16)]
                a = jnp.where(valid, a, 0)
                t = a & (_T - 1)
                pvec = jnp.where(t > 0, a - 1, a)
                hbv = plsc.bitcast(
                    plsc.load_gather(hb_v, [rem8 + lbase + lane]), jnp.uint32)
                lbv = plsc.bitcast(
                    plsc.load_gather(lb_v, [rem8 + lbase + lane]), jnp.uint32)
                r = lax.rem(lax.rem(hbv, span) * mult + lax.rem(lbv, span),
                            span)
                r = r.astype(jnp.int32)
                u = jnp.zeros((16,), jnp.int32)
                for uu in range(_NT):
                    u = u + jnp.where(r >= negoff[uu], 1, 0)
                u = u - 1
                local = r - plsc.load_gather(negoff_v, [u])
                addr = u * _C + local
                ridx_v[...] = addr >> 7
                pltpu.async_copy(negl.at[ridx_v], nidx_v, sem0).wait()
                n_idx = plsc.load_gather(nidx_v, [lane, addr & 127])
                n_idx = jnp.where(valid, n_idx, 0)
                aidx_v[...] = a
                pidx_v[...] = pvec
                gidx_v[...] = n_idx
                pltpu.make_async_copy(ff.at[aidx_v], ab, s1).start()
                pltpu.make_async_copy(ff.at[pidx_v], pb, s2).start()
                pltpu.make_async_copy(ff.at[gidx_v], nb, s3).start()

        def consume_chunk(c, ab, pb, nb, s1, s2, s3, acc):
            @pl.when(c < nchunks)
            def _():
                pltpu.make_async_copy(ff.at[lane], ab, s1).wait()
                pltpu.make_async_copy(ff.at[lane], pb, s2).wait()
                pltpu.make_async_copy(ff.at[lane], nb, s3).wait()
            lbase = c * 16
            valid = lane < (npos_w - lbase)

            def drow(j, _):
                sa = jnp.zeros((16,), jnp.float32)
                sb = jnp.zeros((16,), jnp.float32)
                for k in range(_D // 16):
                    av = ab[j, pl.ds(k * 16, 16)]
                    pv = pb[j, pl.ds(k * 16, 16)]
                    nv = nb[j, pl.ds(k * 16, 16)]
                    da = av - pv + _EPS
                    db = av - nv + _EPS
                    sa = sa + da * da
                    sb = sb + db * db
                d2sa[pl.ds(j * 16, 16)] = sa
                d2sb[pl.ds(j * 16, 16)] = sb
                return 0

            lax.fori_loop(0, 16, drow, 0)
            d2a = jnp.zeros((16,), jnp.float32)
            d2b = jnp.zeros((16,), jnp.float32)
            for col in range(16):
                d2a = d2a + plsc.load_gather(d2sa, [lane * 16 + col])
                d2b = d2b + plsc.load_gather(d2sb, [lane * 16 + col])
            dap = _sqrt16(d2a)
            dan = _sqrt16(d2b)
            per = jnp.maximum(dap - dan + _MARGIN, 0.0)
            return acc + jnp.where(valid, per, 0.0)

        start_chunk(0, aidx_va, pidx_va, gidx_va,
                    abuf_a, pbuf_a, nbuf_a, sem1a, sem2a, sem3a)

        def gbody(g, acc):
            c0 = 2 * g
            start_chunk(c0 + 1, aidx_vb, pidx_vb, gidx_vb,
                        abuf_b, pbuf_b, nbuf_b, sem1b, sem2b, sem3b)
            acc = consume_chunk(c0, abuf_a, pbuf_a, nbuf_a,
                                sem1a, sem2a, sem3a, acc)
            start_chunk(c0 + 2, aidx_va, pidx_va, gidx_va,
                        abuf_a, pbuf_a, nbuf_a, sem1a, sem2a, sem3a)
            acc = consume_chunk(c0 + 1, abuf_b, pbuf_b, nbuf_b,
                                sem1b, sem2b, sem3b, acc)
            return acc

        npairs = (nchunks + 1) // 2
        acc = lax.fori_loop(0, npairs, gbody, jnp.zeros((16,), jnp.float32))

        outf_v[...] = jnp.full((16,), jnp.sum(acc), jnp.float32)
        pltpu.sync_copy(outf_v, part_s.at[pl.ds(w * 16, 16)])
        plsc.subcore_barrier()

        @pl.when(w == 0)
        def _():
            pltpu.sync_copy(part_s, pf_v)
            tot = jnp.sum(plsc.load_gather(pf_v, [lane * 17]))
            totv = jnp.full((16,), tot, jnp.float32)
            denv = jnp.full((16,), jnp.maximum(n_pos, 1).astype(jnp.float32),
                            jnp.float32)
            lossv = totv / denv
            fallback = (n_pos == 0) | (n_neg == 0)
            out_v[...] = jnp.where(fallback, jnp.float32(1e-6), lossv)
            pltpu.sync_copy(out_v, out)


def _make_call():
    mesh = plsc.VectorSubcoreMesh(core_axis_name="c", subcore_axis_name="s",
                                  num_cores=2, num_subcores=16)
    return pl.kernel(
        _body,
        out_type=(jax.ShapeDtypeStruct((16,), jnp.float32),
                  jax.ShapeDtypeStruct((_TOTAL // 128, 128), jnp.int32)),
        mesh=mesh,
        compiler_params=pltpu.CompilerParams(needs_layout_passes=False),
        scratch_types=[
            pltpu.VMEM((2064,), jnp.int32),
            pltpu.VMEM((2048,), jnp.float32),
            pltpu.VMEM((2064,), jnp.int32),
            pltpu.VMEM((17, 128), jnp.int32),
            pltpu.VMEM((2064,), jnp.int32),
            pltpu.VMEM((2064,), jnp.int32),
            pltpu.VMEM((16,), jnp.int32),
            pltpu.VMEM((256,), jnp.int32),
            pltpu.VMEM((16,), jnp.int32),
            pltpu.VMEM((16, 128), jnp.int32),
            pltpu.VMEM((16,), jnp.int32),
            pltpu.VMEM((16,), jnp.int32),
            pltpu.VMEM((16,), jnp.int32),
            pltpu.VMEM((16,), jnp.int32),
            pltpu.VMEM((16,), jnp.int32),
            pltpu.VMEM((16,), jnp.int32),
            pltpu.VMEM((16,), jnp.int32),
            pltpu.VMEM((16, _D), jnp.float32),
            pltpu.VMEM((16, _D), jnp.float32),
            pltpu.VMEM((16, _D), jnp.float32),
            pltpu.VMEM((16, _D), jnp.float32),
            pltpu.VMEM((16, _D), jnp.float32),
            pltpu.VMEM((16, _D), jnp.float32),
            pltpu.VMEM((256,), jnp.float32),
            pltpu.VMEM((256,), jnp.float32),
            pltpu.VMEM((16,), jnp.float32),
            pltpu.VMEM((16,), jnp.float32),
            pltpu.VMEM((256,), jnp.float32),
            pltpu.VMEM_SHARED((256,), jnp.int32),
            pltpu.VMEM_SHARED((256,), jnp.float32),
            pltpu.SemaphoreType.DMA,
            pltpu.SemaphoreType.DMA,
            pltpu.SemaphoreType.DMA,
            pltpu.SemaphoreType.DMA,
            pltpu.SemaphoreType.DMA,
            pltpu.SemaphoreType.DMA,
            pltpu.SemaphoreType.DMA,
        ],
    )


_sc_call = _make_call()


def kernel(frame_features, boundaries, ph_labels):
    ff = frame_features.reshape(_TOTAL, _D)
    k1, k2 = jax.random.split(jax.random.key(123))
    zpad = jnp.zeros((_BITS_PAD,), jnp.uint32)
    hb = lax.bitcast_convert_type(
        jnp.concatenate([jax.random.bits(k1, (_TOTAL,), jnp.uint32), zpad]),
        jnp.int32)
    lb = lax.bitcast_convert_type(
        jnp.concatenate([jax.random.bits(k2, (_TOTAL,), jnp.uint32), zpad]),
        jnp.int32)
    out, _ = _sc_call(ff, boundaries, ph_labels.astype(jnp.int32), hb, lb)
    return out[0]
